# baseline (speedup 1.0000x reference)
import functools

import jax
import jax.numpy as jnp
from jax import lax
from jax.experimental import pallas as pl
from jax.experimental.pallas import tpu as pltpu
from jax.experimental.pallas import tpu_sc as plsc

N = 10000
E = 320000
D = 128
C = 40
ZR = 640

NC, NS = 2, 16
NW = NC * NS
CH = 125
NCH_T = E // CH
NCH_W = NCH_T // NW
G = 16
NG = NCH_W // G
SA = 632
SL = N - (NS - 1) * SA

_mesh = plsc.VectorSubcoreMesh(core_axis_name="c", subcore_axis_name="s",
                               num_cores=NC, num_subcores=NS)


def _each_stripe(s, fn):
    @pl.when(s < NS - 1)
    def _main():
        fn(s * SA, SA)

    @pl.when(s == NS - 1)
    def _last():
        fn((NS - 1) * SA, SL)


NP1 = 10240
RPS = NP1 // NS


@functools.partial(
    pl.kernel,
    out_type=(jax.ShapeDtypeStruct((NP1,), jnp.float32),
              jax.ShapeDtypeStruct((NP1,), jnp.float32)),
    mesh=_mesh,
    scratch_types=[
        pltpu.VMEM((NCH_W, CH), jnp.int32),
        pltpu.VMEM((CH,), jnp.float32),
        pltpu.VMEM_SHARED((NP1,), jnp.float32),
    ],
)
def _sc_degree(col_hbm, ones_hbm, zeros_hbm, out0_hbm, out1_hbm,
               cidx_v, ones_v, hist_sh):
    c = lax.axis_index("c")
    s = lax.axis_index("s")
    wid = s * NC + c
    pltpu.sync_copy(col_hbm.at[pl.ds(wid * NCH_W, NCH_W)], cidx_v)
    pltpu.sync_copy(ones_hbm.at[pl.ds(0, CH)], ones_v)
    pltpu.sync_copy(zeros_hbm, hist_sh.at[pl.ds(s * RPS, RPS)])
    plsc.subcore_barrier()

    def step(j, carry):
        pltpu.sync_copy(ones_v, hist_sh.at[cidx_v.at[j]], add=True)
        return carry

    lax.fori_loop(0, NCH_W, step, 0)
    plsc.subcore_barrier()

    @pl.when(c == 0)
    def _w0():
        pltpu.sync_copy(hist_sh.at[pl.ds(s * RPS, RPS)],
                        out0_hbm.at[pl.ds(s * RPS, RPS)])

    @pl.when(c == 1)
    def _w1():
        pltpu.sync_copy(hist_sh.at[pl.ds(s * RPS, RPS)],
                        out1_hbm.at[pl.ds(s * RPS, RPS)])


@functools.partial(
    pl.kernel,
    out_type=jax.ShapeDtypeStruct((NC, N, D), jnp.float32),
    mesh=_mesh,
    scratch_types=[
        pltpu.VMEM((2, G, CH), jnp.int32),
        pltpu.VMEM((2, G, CH), jnp.int32),
        pltpu.VMEM((CH, D), jnp.float32),
        pltpu.VMEM((CH, D), jnp.float32),
        pltpu.VMEM_SHARED((N, D), jnp.float32),
        pltpu.SemaphoreType.DMA,
        pltpu.SemaphoreType.DMA,
        pltpu.SemaphoreType.DMA,
    ],
)
def _sc_prop(z_hbm, row_hbm, col_hbm, zeros_hbm, out_hbm,
             ridx_v, cidx_v, rows0_v, rows1_v, acc_sh, sem0, sem1, isem):
    c = lax.axis_index("c")
    s = lax.axis_index("s")
    wid = s * NC + c
    base = wid * NCH_W
    pltpu.sync_copy(row_hbm.at[pl.ds(base, G)], ridx_v.at[0])
    pltpu.sync_copy(col_hbm.at[pl.ds(base, G)], cidx_v.at[0])

    @pl.when(c == 0)
    def _initz():
        _each_stripe(s, lambda b, n: pltpu.sync_copy(
            z_hbm.at[pl.ds(b, n)], acc_sh.at[pl.ds(b, n)]))

    @pl.when(c == 1)
    def _init0():
        _each_stripe(s, lambda b, n: pltpu.sync_copy(
            zeros_hbm.at[pl.ds(0, n)], acc_sh.at[pl.ds(b, n)]))

    plsc.subcore_barrier()

    for g in range(NG):
        slot, nxt = g % 2, (g + 1) % 2
        if g + 1 < NG:
            gb = base + (g + 1) * G
            pltpu.async_copy(row_hbm.at[pl.ds(gb, G)], ridx_v.at[nxt], isem)
            pltpu.async_copy(col_hbm.at[pl.ds(gb, G)], cidx_v.at[nxt], isem)
        pltpu.async_copy(z_hbm.at[ridx_v.at[slot, 0]], rows0_v, sem0)

        def step(i, carry, slot=slot):
            j = 2 * i
            cp1 = pltpu.async_copy(
                z_hbm.at[ridx_v.at[slot, j + 1]], rows1_v, sem1)
            pltpu.make_async_copy(
                z_hbm.at[ridx_v.at[slot, j]], rows0_v, sem0).wait()
            pltpu.sync_copy(rows0_v, acc_sh.at[cidx_v.at[slot, j]], add=True)

            @pl.when(j + 2 < G)
            def _prefetch():
                pltpu.async_copy(
                    z_hbm.at[ridx_v.at[slot, j + 2]], rows0_v, sem0)

            cp1.wait()
            pltpu.sync_copy(rows1_v, acc_sh.at[cidx_v.at[slot, j + 1]],
                            add=True)
            return carry

        lax.fori_loop(0, G // 2, step, 0)
        if g + 1 < NG:
            gb = base + (g + 1) * G
            pltpu.make_async_copy(
                row_hbm.at[pl.ds(gb, G)], ridx_v.at[nxt], isem).wait()
            pltpu.make_async_copy(
                col_hbm.at[pl.ds(gb, G)], cidx_v.at[nxt], isem).wait()
    plsc.subcore_barrier()
    _each_stripe(s, lambda b, n: pltpu.sync_copy(
        acc_sh.at[pl.ds(b, n)], out_hbm.at[c, pl.ds(b, n)]))


R = 2000


def _tc_mm_body(x, w1, xw):
    xw[...] = jnp.dot(x[...], w1[...], preferred_element_type=jnp.float32)


_tc_mm = pl.pallas_call(
    _tc_mm_body,
    grid=(N // R,),
    in_specs=[
        pl.BlockSpec((R, D), lambda i: (i, 0)),
        pl.BlockSpec((D, D), lambda i: (0, 0)),
    ],
    out_specs=pl.BlockSpec((R, D), lambda i: (i, 0)),
    out_shape=jax.ShapeDtypeStruct((N, D), jnp.float32),
)


def _tc1_body(h0, h1, xw, z, dis):
    dis_v = lax.rsqrt(h0[...] + h1[...] + 1.0)
    dis[...] = dis_v
    z[...] = dis_v * xw[...]


_tc1 = pl.pallas_call(
    _tc1_body,
    grid=(N // R,),
    in_specs=[
        pl.BlockSpec((R, 1), lambda i: (i, 0)),
        pl.BlockSpec((R, 1), lambda i: (i, 0)),
        pl.BlockSpec((R, D), lambda i: (i, 0)),
    ],
    out_specs=[
        pl.BlockSpec((R, D), lambda i: (i, 0)),
        pl.BlockSpec((R, 1), lambda i: (i, 0)),
    ],
    out_shape=[
        jax.ShapeDtypeStruct((N, D), jnp.float32),
        jax.ShapeDtypeStruct((N, 1), jnp.float32),
    ],
)


def _tc2_body(p, dis, b1, z2):
    h = jnp.maximum(dis[...] * (p[0] + p[1]) + b1[...], 0.0)
    z2[...] = dis[...] * h


_tc2 = pl.pallas_call(
    _tc2_body,
    grid=(N // R,),
    in_specs=[
        pl.BlockSpec((2, R, D), lambda i: (0, i, 0)),
        pl.BlockSpec((R, 1), lambda i: (i, 0)),
        pl.BlockSpec((1, D), lambda i: (0, 0)),
    ],
    out_specs=pl.BlockSpec((R, D), lambda i: (i, 0)),
    out_shape=jax.ShapeDtypeStruct((N, D), jnp.float32),
)


def _tc3_body(p, dis, w2, b2, out):
    g = dis[...] * (p[0] + p[1])
    o = jnp.dot(g, w2[...], preferred_element_type=jnp.float32) + b2[...]
    mx = jnp.max(o, axis=1, keepdims=True)
    ssum = jnp.sum(jnp.exp(o - mx), axis=1, keepdims=True)
    out[...] = (o - mx) - jnp.log(ssum)


_tc3 = pl.pallas_call(
    _tc3_body,
    grid=(N // R,),
    in_specs=[
        pl.BlockSpec((2, R, D), lambda i: (0, i, 0)),
        pl.BlockSpec((R, 1), lambda i: (i, 0)),
        pl.BlockSpec((D, C), lambda i: (0, 0)),
        pl.BlockSpec((1, C), lambda i: (0, 0)),
    ],
    out_specs=pl.BlockSpec((R, C), lambda i: (i, 0)),
    out_shape=jax.ShapeDtypeStruct((N, C), jnp.float32),
)


def kernel(x, edge_index, W1, b1, W2, b2):
    col2d = edge_index[1].astype(jnp.int32).reshape(NCH_T, CH)
    row2d = lax.optimization_barrier(
        edge_index)[0].astype(jnp.int32).reshape(NCH_T, CH)

    zeros1 = jnp.zeros((ZR,), jnp.float32)
    ones1 = jnp.ones((CH,), jnp.float32)
    zeros_d = jnp.zeros((ZR, D), jnp.float32)
    b1r = b1[None, :]
    b2r = b2[None, :]

    hist0, hist1 = _sc_degree(col2d, ones1, zeros1)
    xw = _tc_mm(x, W1)
    h0 = hist0[:N, None]
    h1 = hist1[:N, None]
    z1, dis = _tc1(h0, h1, xw)
    p1 = _sc_prop(z1, row2d, col2d, zeros_d)
    z2 = _tc2(p1, dis, b1r)
    p2 = _sc_prop(z2, row2d, col2d, zeros_d)
    return _tc3(p2, dis, W2, b2r)


# speedup vs baseline: 1.0219x; 1.0219x over previous
import functools

import jax
import jax.numpy as jnp
from jax import lax
from jax.experimental import pallas as pl
from jax.experimental.pallas import tpu as pltpu
from jax.experimental.pallas import tpu_sc as plsc

N = 10000
E = 320000
D = 128
C = 40
ZR = 640

NC, NS = 2, 16
NW = NC * NS
CH = 125
NCH_T = E // CH
NCH_W = NCH_T // NW
G = 16
NG = NCH_W // G
SA = 632
SL = N - (NS - 1) * SA

_mesh = plsc.VectorSubcoreMesh(core_axis_name="c", subcore_axis_name="s",
                               num_cores=NC, num_subcores=NS)


def _each_stripe(s, fn):
    @pl.when(s < NS - 1)
    def _main():
        fn(s * SA, SA)

    @pl.when(s == NS - 1)
    def _last():
        fn((NS - 1) * SA, SL)


NP1 = 10240
RPS = NP1 // NS


@functools.partial(
    pl.kernel,
    out_type=(jax.ShapeDtypeStruct((NP1,), jnp.float32),
              jax.ShapeDtypeStruct((NP1,), jnp.float32)),
    mesh=_mesh,
    scratch_types=[
        pltpu.VMEM((NCH_W, CH), jnp.int32),
        pltpu.VMEM((CH,), jnp.float32),
        pltpu.VMEM_SHARED((NP1,), jnp.float32),
    ],
)
def _sc_degree(col_hbm, ones_hbm, zeros_hbm, out0_hbm, out1_hbm,
               cidx_v, ones_v, hist_sh):
    c = lax.axis_index("c")
    s = lax.axis_index("s")
    wid = s * NC + c
    pltpu.sync_copy(col_hbm.at[pl.ds(wid * NCH_W, NCH_W)], cidx_v)
    pltpu.sync_copy(ones_hbm.at[pl.ds(0, CH)], ones_v)
    pltpu.sync_copy(zeros_hbm, hist_sh.at[pl.ds(s * RPS, RPS)])
    plsc.subcore_barrier()

    def step(j, carry):
        pltpu.sync_copy(ones_v, hist_sh.at[cidx_v.at[j]], add=True)
        return carry

    lax.fori_loop(0, NCH_W, step, 0)
    plsc.subcore_barrier()

    @pl.when(c == 0)
    def _w0():
        pltpu.sync_copy(hist_sh.at[pl.ds(s * RPS, RPS)],
                        out0_hbm.at[pl.ds(s * RPS, RPS)])

    @pl.when(c == 1)
    def _w1():
        pltpu.sync_copy(hist_sh.at[pl.ds(s * RPS, RPS)],
                        out1_hbm.at[pl.ds(s * RPS, RPS)])


@functools.partial(
    pl.kernel,
    out_type=jax.ShapeDtypeStruct((NC, N, D), jnp.float32),
    mesh=_mesh,
    scratch_types=[
        pltpu.VMEM((2, G, CH), jnp.int32),
        pltpu.VMEM((2, G, CH), jnp.int32),
        pltpu.VMEM((CH, D), jnp.float32),
        pltpu.VMEM((CH, D), jnp.float32),
        pltpu.VMEM_SHARED((N, D), jnp.float32),
        pltpu.SemaphoreType.DMA,
        pltpu.SemaphoreType.DMA,
        pltpu.SemaphoreType.DMA,
    ],
)
def _sc_prop(z_hbm, row_hbm, col_hbm, zeros_hbm, out_hbm,
             ridx_v, cidx_v, rows0_v, rows1_v, acc_sh, sem0, sem1, isem):
    c = lax.axis_index("c")
    s = lax.axis_index("s")
    wid = s * NC + c
    base = wid * NCH_W
    pltpu.sync_copy(row_hbm.at[pl.ds(base, G)], ridx_v.at[0])
    pltpu.sync_copy(col_hbm.at[pl.ds(base, G)], cidx_v.at[0])

    @pl.when(c == 0)
    def _initz():
        _each_stripe(s, lambda b, n: pltpu.sync_copy(
            z_hbm.at[pl.ds(b, n)], acc_sh.at[pl.ds(b, n)]))

    @pl.when(c == 1)
    def _init0():
        _each_stripe(s, lambda b, n: pltpu.sync_copy(
            zeros_hbm.at[pl.ds(0, n)], acc_sh.at[pl.ds(b, n)]))

    plsc.subcore_barrier()

    for g in range(NG):
        slot, nxt = g % 2, (g + 1) % 2
        if g + 1 < NG:
            gb = base + (g + 1) * G
            pltpu.async_copy(row_hbm.at[pl.ds(gb, G)], ridx_v.at[nxt], isem)
            pltpu.async_copy(col_hbm.at[pl.ds(gb, G)], cidx_v.at[nxt], isem)
        pltpu.async_copy(z_hbm.at[ridx_v.at[slot, 0]], rows0_v, sem0)

        def step(i, carry, slot=slot):
            j = 2 * i
            cp1 = pltpu.async_copy(
                z_hbm.at[ridx_v.at[slot, j + 1]], rows1_v, sem1)
            pltpu.make_async_copy(
                z_hbm.at[ridx_v.at[slot, j]], rows0_v, sem0).wait()
            pltpu.sync_copy(rows0_v, acc_sh.at[cidx_v.at[slot, j]], add=True)

            @pl.when(j + 2 < G)
            def _prefetch():
                pltpu.async_copy(
                    z_hbm.at[ridx_v.at[slot, j + 2]], rows0_v, sem0)

            cp1.wait()
            pltpu.sync_copy(rows1_v, acc_sh.at[cidx_v.at[slot, j + 1]],
                            add=True)
            return carry

        lax.fori_loop(0, G // 2, step, 0)
        if g + 1 < NG:
            gb = base + (g + 1) * G
            pltpu.make_async_copy(
                row_hbm.at[pl.ds(gb, G)], ridx_v.at[nxt], isem).wait()
            pltpu.make_async_copy(
                col_hbm.at[pl.ds(gb, G)], cidx_v.at[nxt], isem).wait()
    plsc.subcore_barrier()
    _each_stripe(s, lambda b, n: pltpu.sync_copy(
        acc_sh.at[pl.ds(b, n)], out_hbm.at[c, pl.ds(b, n)]))


R = 2000


def _tc_mm_body(x, w1, xw):
    xw[...] = jnp.dot(x[...], w1[...], preferred_element_type=jnp.float32)


_tc_mm = pl.pallas_call(
    _tc_mm_body,
    grid=(N // R,),
    in_specs=[
        pl.BlockSpec((R, D), lambda i: (i, 0)),
        pl.BlockSpec((D, D), lambda i: (0, 0)),
    ],
    out_specs=pl.BlockSpec((R, D), lambda i: (i, 0)),
    out_shape=jax.ShapeDtypeStruct((N, D), jnp.float32),
)


def _tc1_body(dg, xw, z):
    z[...] = lax.rsqrt(dg[...] + 1.0) * xw[...]


_tc1 = pl.pallas_call(
    _tc1_body,
    grid=(N // R,),
    in_specs=[
        pl.BlockSpec((R, 1), lambda i: (i, 0)),
        pl.BlockSpec((R, D), lambda i: (i, 0)),
    ],
    out_specs=pl.BlockSpec((R, D), lambda i: (i, 0)),
    out_shape=jax.ShapeDtypeStruct((N, D), jnp.float32),
)


def _tc2_body(p, dg, b1, z2):
    dis = lax.rsqrt(dg[...] + 1.0)
    h = jnp.maximum(dis * (p[0] + p[1]) + b1[...], 0.0)
    z2[...] = dis * h


_tc2 = pl.pallas_call(
    _tc2_body,
    grid=(N // R,),
    in_specs=[
        pl.BlockSpec((2, R, D), lambda i: (0, i, 0)),
        pl.BlockSpec((R, 1), lambda i: (i, 0)),
        pl.BlockSpec((1, D), lambda i: (0, 0)),
    ],
    out_specs=pl.BlockSpec((R, D), lambda i: (i, 0)),
    out_shape=jax.ShapeDtypeStruct((N, D), jnp.float32),
)


def _tc3_body(p, dg, w2, b2, out):
    g = lax.rsqrt(dg[...] + 1.0) * (p[0] + p[1])
    o = jnp.dot(g, w2[...], preferred_element_type=jnp.float32) + b2[...]
    mx = jnp.max(o, axis=1, keepdims=True)
    ssum = jnp.sum(jnp.exp(o - mx), axis=1, keepdims=True)
    out[...] = (o - mx) - jnp.log(ssum)


_tc3 = pl.pallas_call(
    _tc3_body,
    grid=(N // R,),
    in_specs=[
        pl.BlockSpec((2, R, D), lambda i: (0, i, 0)),
        pl.BlockSpec((R, 1), lambda i: (i, 0)),
        pl.BlockSpec((D, C), lambda i: (0, 0)),
        pl.BlockSpec((1, C), lambda i: (0, 0)),
    ],
    out_specs=pl.BlockSpec((R, C), lambda i: (i, 0)),
    out_shape=jax.ShapeDtypeStruct((N, C), jnp.float32),
)


def kernel(x, edge_index, W1, b1, W2, b2):
    col2d = edge_index[1].astype(jnp.int32).reshape(NCH_T, CH)
    row2d = lax.optimization_barrier(
        edge_index)[0].astype(jnp.int32).reshape(NCH_T, CH)

    zeros1 = jnp.zeros((ZR,), jnp.float32)
    ones1 = jnp.ones((CH,), jnp.float32)
    zeros_d = jnp.zeros((ZR, D), jnp.float32)
    b1r = b1[None, :]
    b2r = b2[None, :]

    hist0, hist1 = _sc_degree(col2d, ones1, zeros1)
    xw = _tc_mm(x, W1)
    dg = (hist0 + hist1)[:N, None]
    z1 = _tc1(dg, xw)
    p1 = _sc_prop(z1, row2d, col2d, zeros_d)
    z2 = _tc2(p1, dg, b1r)
    p2 = _sc_prop(z2, row2d, col2d, zeros_d)
    return _tc3(p2, dg, W2, b2r)


# speedup vs baseline: 1.0551x; 1.0326x over previous
import functools

import jax
import jax.numpy as jnp
from jax import lax
from jax.experimental import pallas as pl
from jax.experimental.pallas import tpu as pltpu
from jax.experimental.pallas import tpu_sc as plsc

N = 10000
E = 320000
D = 128
C = 40
ZR = 640

NC, NS = 2, 16
NW = NC * NS
CH = 125
NCH_T = E // CH
NCH_W = NCH_T // NW
G = 16
NG = NCH_W // G
SA = 632
SL = N - (NS - 1) * SA

_mesh = plsc.VectorSubcoreMesh(core_axis_name="c", subcore_axis_name="s",
                               num_cores=NC, num_subcores=NS)


def _each_stripe(s, fn):
    @pl.when(s < NS - 1)
    def _main():
        fn(s * SA, SA)

    @pl.when(s == NS - 1)
    def _last():
        fn((NS - 1) * SA, SL)


NP1 = 10240
RPS = NP1 // NS


@functools.partial(
    pl.kernel,
    out_type=(jax.ShapeDtypeStruct((NP1,), jnp.float32),
              jax.ShapeDtypeStruct((NP1,), jnp.float32)),
    mesh=_mesh,
    scratch_types=[
        pltpu.VMEM((NCH_W, CH), jnp.int32),
        pltpu.VMEM((CH,), jnp.float32),
        pltpu.VMEM_SHARED((NP1,), jnp.float32),
    ],
)
def _sc_degree(col_hbm, ones_hbm, zeros_hbm, out0_hbm, out1_hbm,
               cidx_v, ones_v, hist_sh):
    c = lax.axis_index("c")
    s = lax.axis_index("s")
    wid = s * NC + c
    pltpu.sync_copy(col_hbm.at[pl.ds(wid * NCH_W, NCH_W)], cidx_v)
    pltpu.sync_copy(ones_hbm.at[pl.ds(0, CH)], ones_v)
    pltpu.sync_copy(zeros_hbm, hist_sh.at[pl.ds(s * RPS, RPS)])
    plsc.subcore_barrier()

    def step(j, carry):
        pltpu.sync_copy(ones_v, hist_sh.at[cidx_v.at[j]], add=True)
        return carry

    lax.fori_loop(0, NCH_W, step, 0)
    plsc.subcore_barrier()

    @pl.when(c == 0)
    def _w0():
        pltpu.sync_copy(hist_sh.at[pl.ds(s * RPS, RPS)],
                        out0_hbm.at[pl.ds(s * RPS, RPS)])

    @pl.when(c == 1)
    def _w1():
        pltpu.sync_copy(hist_sh.at[pl.ds(s * RPS, RPS)],
                        out1_hbm.at[pl.ds(s * RPS, RPS)])


@functools.partial(
    pl.kernel,
    out_type=jax.ShapeDtypeStruct((NC, N, D), jnp.float32),
    mesh=_mesh,
    scratch_types=[
        pltpu.VMEM((2, G, CH), jnp.int32),
        pltpu.VMEM((2, G, CH), jnp.int32),
        pltpu.VMEM((CH, D), jnp.float32),
        pltpu.VMEM((CH, D), jnp.float32),
        pltpu.VMEM_SHARED((N, D), jnp.float32),
        pltpu.SemaphoreType.DMA,
        pltpu.SemaphoreType.DMA,
        pltpu.SemaphoreType.DMA,
    ],
)
def _sc_prop(z_hbm, row_hbm, col_hbm, zeros_hbm, out_hbm,
             ridx_v, cidx_v, rows0_v, rows1_v, acc_sh, sem0, sem1, isem):
    c = lax.axis_index("c")
    s = lax.axis_index("s")
    wid = s * NC + c
    base = wid * NCH_W
    pltpu.sync_copy(row_hbm.at[pl.ds(base, G)], ridx_v.at[0])
    pltpu.sync_copy(col_hbm.at[pl.ds(base, G)], cidx_v.at[0])

    @pl.when(c == 0)
    def _initz():
        _each_stripe(s, lambda b, n: pltpu.sync_copy(
            z_hbm.at[pl.ds(b, n)], acc_sh.at[pl.ds(b, n)]))

    @pl.when(c == 1)
    def _init0():
        _each_stripe(s, lambda b, n: pltpu.sync_copy(
            zeros_hbm.at[pl.ds(0, n)], acc_sh.at[pl.ds(b, n)]))

    plsc.subcore_barrier()

    for g in range(NG):
        slot, nxt = g % 2, (g + 1) % 2
        if g + 1 < NG:
            gb = base + (g + 1) * G
            pltpu.async_copy(row_hbm.at[pl.ds(gb, G)], ridx_v.at[nxt], isem)
            pltpu.async_copy(col_hbm.at[pl.ds(gb, G)], cidx_v.at[nxt], isem)
        pltpu.async_copy(z_hbm.at[ridx_v.at[slot, 0]], rows0_v, sem0)

        def step(i, carry, slot=slot):
            j = 2 * i
            cp1 = pltpu.async_copy(
                z_hbm.at[ridx_v.at[slot, j + 1]], rows1_v, sem1)
            pltpu.make_async_copy(
                z_hbm.at[ridx_v.at[slot, j]], rows0_v, sem0).wait()
            pltpu.sync_copy(rows0_v, acc_sh.at[cidx_v.at[slot, j]], add=True)

            @pl.when(j + 2 < G)
            def _prefetch():
                pltpu.async_copy(
                    z_hbm.at[ridx_v.at[slot, j + 2]], rows0_v, sem0)

            cp1.wait()
            pltpu.sync_copy(rows1_v, acc_sh.at[cidx_v.at[slot, j + 1]],
                            add=True)
            return carry

        lax.fori_loop(0, G // 2, step, 0)
        if g + 1 < NG:
            gb = base + (g + 1) * G
            pltpu.make_async_copy(
                row_hbm.at[pl.ds(gb, G)], ridx_v.at[nxt], isem).wait()
            pltpu.make_async_copy(
                col_hbm.at[pl.ds(gb, G)], cidx_v.at[nxt], isem).wait()
    plsc.subcore_barrier()
    _each_stripe(s, lambda b, n: pltpu.sync_copy(
        acc_sh.at[pl.ds(b, n)], out_hbm.at[c, pl.ds(b, n)]))


R = 2000


def _tc_mm_body(x, w1, xw):
    xw[...] = jnp.dot(x[...], w1[...], preferred_element_type=jnp.float32)


_tc_mm = pl.pallas_call(
    _tc_mm_body,
    grid=(N // R,),
    in_specs=[
        pl.BlockSpec((R, D), lambda i: (i, 0)),
        pl.BlockSpec((D, D), lambda i: (0, 0)),
    ],
    out_specs=pl.BlockSpec((R, D), lambda i: (i, 0)),
    out_shape=jax.ShapeDtypeStruct((N, D), jnp.float32),
)


def _tc1_body(dg, xw, z):
    z[...] = lax.rsqrt(dg[...] + 1.0) * xw[...]


_tc1 = pl.pallas_call(
    _tc1_body,
    grid=(N // R,),
    in_specs=[
        pl.BlockSpec((R, 1), lambda i: (i, 0)),
        pl.BlockSpec((R, D), lambda i: (i, 0)),
    ],
    out_specs=pl.BlockSpec((R, D), lambda i: (i, 0)),
    out_shape=jax.ShapeDtypeStruct((N, D), jnp.float32),
)


def _tc2_body(p, dg, b1, z2):
    dis = lax.rsqrt(dg[...] + 1.0)
    h = jnp.maximum(dis * (p[0] + p[1]) + b1[...], 0.0)
    z2[...] = dis * h


_tc2 = pl.pallas_call(
    _tc2_body,
    grid=(N // R,),
    in_specs=[
        pl.BlockSpec((2, R, D), lambda i: (0, i, 0)),
        pl.BlockSpec((R, 1), lambda i: (i, 0)),
        pl.BlockSpec((1, D), lambda i: (0, 0)),
    ],
    out_specs=pl.BlockSpec((R, D), lambda i: (i, 0)),
    out_shape=jax.ShapeDtypeStruct((N, D), jnp.float32),
)


def _tc3_body(p, dg, w2, b2, out):
    g = lax.rsqrt(dg[...] + 1.0) * (p[0] + p[1])
    o = jnp.dot(g, w2[...], preferred_element_type=jnp.float32) + b2[...]
    mx = jnp.max(o, axis=1, keepdims=True)
    ssum = jnp.sum(jnp.exp(o - mx), axis=1, keepdims=True)
    out[...] = (o - mx) - jnp.log(ssum)


_tc3 = pl.pallas_call(
    _tc3_body,
    grid=(N // R,),
    in_specs=[
        pl.BlockSpec((2, R, D), lambda i: (0, i, 0)),
        pl.BlockSpec((R, 1), lambda i: (i, 0)),
        pl.BlockSpec((D, C), lambda i: (0, 0)),
        pl.BlockSpec((1, C), lambda i: (0, 0)),
    ],
    out_specs=pl.BlockSpec((R, C), lambda i: (i, 0)),
    out_shape=jax.ShapeDtypeStruct((N, C), jnp.float32),
)


def kernel(x, edge_index, W1, b1, W2, b2):
    col2d = edge_index[1].astype(jnp.int32).reshape(NCH_T, CH)
    row2d = edge_index[0].astype(jnp.int32).reshape(NCH_T, CH)

    zeros1 = jnp.zeros((ZR,), jnp.float32)
    ones1 = jnp.ones((CH,), jnp.float32)
    zeros_d = jnp.zeros((ZR, D), jnp.float32)
    b1r = b1[None, :]
    b2r = b2[None, :]

    hist0, hist1 = _sc_degree(col2d, ones1, zeros1)
    xw = _tc_mm(x, W1)
    dg = (hist0 + hist1)[:N, None]
    z1 = _tc1(dg, xw)
    p1 = _sc_prop(z1, row2d, col2d, zeros_d)
    z2 = _tc2(p1, dg, b1r)
    p2 = _sc_prop(z2, row2d, col2d, zeros_d)
    return _tc3(p2, dg, W2, b2r)
